# Initial kernel scaffold; baseline (speedup 1.0000x reference)
import jax
import jax.numpy as jnp
from jax.experimental import pallas as pl


def kernel(queries, keys):
    raise NotImplementedError("write your pallas kernel here")


# speedup vs baseline: 3.7032x; 3.7032x over previous
import jax
import jax.numpy as jnp
from jax.experimental import pallas as pl
from jax.experimental.pallas import tpu as pltpu

Q_DIM = 784
K_DIM = 100000
D_DIM = 64
KB = 2000
NBLK = K_DIM // KB


def _nn_kernel(q_ref, k_ref, scores_ref, img_ref, idx_ref):
    i = pl.program_id(0)

    @pl.when(i == 0)
    def _init():
        scores_ref[...] = jnp.full((Q_DIM, 1), jnp.inf, dtype=jnp.float32)
        idx_ref[...] = jnp.zeros((Q_DIM, 1), dtype=jnp.int32)

    q = q_ref[...]
    k = k_ref[...]
    s = jax.lax.dot_general(q, k, (((1,), (1,)), ((), ())),
                            preferred_element_type=jnp.float32)
    qsq = jnp.sum(q * q, axis=1, keepdims=True)
    ksq = jnp.sum(k * k, axis=1)
    d = (qsq + ksq[None, :]) - 2.0 * s

    bmin = jnp.min(d, axis=1, keepdims=True)
    gidx = i * KB + jax.lax.broadcasted_iota(jnp.int32, (1, KB), 1)
    barg = jnp.min(jnp.where(d == bmin, gidx, K_DIM),
                   axis=1, keepdims=True)

    cur = scores_ref[...]
    better = bmin < cur
    newv = jnp.where(better, bmin, cur)
    idx_ref[...] = jnp.where(better, barg, idx_ref[...])

    is_last = i == NBLK - 1

    @pl.when(jnp.logical_not(is_last))
    def _acc():
        scores_ref[...] = newv

    @pl.when(is_last)
    def _fin():
        ps = jnp.sqrt(jnp.maximum(newv, 0.0) + 1e-12)
        scores_ref[...] = ps
        img_ref[...] = jnp.max(ps, axis=(0, 1), keepdims=True)


def kernel(queries, keys):
    scores, img, idx = pl.pallas_call(
        _nn_kernel,
        grid=(NBLK,),
        in_specs=[
            pl.BlockSpec((Q_DIM, D_DIM), lambda i: (0, 0)),
            pl.BlockSpec((KB, D_DIM), lambda i: (i, 0)),
        ],
        out_specs=[
            pl.BlockSpec((Q_DIM, 1), lambda i: (0, 0)),
            pl.BlockSpec((1, 1), lambda i: (0, 0)),
            pl.BlockSpec((Q_DIM, 1), lambda i: (0, 0)),
        ],
        out_shape=[
            jax.ShapeDtypeStruct((Q_DIM, 1), jnp.float32),
            jax.ShapeDtypeStruct((1, 1), jnp.float32),
            jax.ShapeDtypeStruct((Q_DIM, 1), jnp.int32),
        ],
        compiler_params=pltpu.CompilerParams(
            dimension_semantics=("arbitrary",),
        ),
    )(queries, keys)
    return scores[:, 0], img[0, 0], idx[:, 0]


# speedup vs baseline: 4.0832x; 1.1026x over previous
import jax
import jax.numpy as jnp
from jax.experimental import pallas as pl
from jax.experimental.pallas import tpu as pltpu

Q_DIM = 784
K_DIM = 100000
D_DIM = 64
KB = 2000
NBLK = K_DIM // KB


def _nn_kernel(q_ref, k_ref, scores_ref, img_ref, idx_ref):
    i = pl.program_id(0)

    @pl.when(i == 0)
    def _init():
        scores_ref[...] = jnp.full((Q_DIM, 1), jnp.inf, dtype=jnp.float32)
        idx_ref[...] = jnp.zeros((Q_DIM, 1), dtype=jnp.int32)

    q = q_ref[...]
    k = k_ref[...]
    ksq = jnp.sum(k * k, axis=1, keepdims=True)
    khat = jnp.concatenate([k, ksq], axis=1)
    qhat = jnp.concatenate([q * -2.0, jnp.ones((Q_DIM, 1), jnp.float32)],
                           axis=1)
    a = jax.lax.dot_general(qhat, khat, (((1,), (1,)), ((), ())),
                            preferred_element_type=jnp.float32)

    bmin = jnp.min(a, axis=1, keepdims=True)
    gidx = i * KB + jax.lax.broadcasted_iota(jnp.int32, (1, KB), 1)
    barg = jnp.min(jnp.where(a == bmin, gidx, K_DIM),
                   axis=1, keepdims=True)

    cur = scores_ref[...]
    better = bmin < cur
    newv = jnp.where(better, bmin, cur)
    idx_ref[...] = jnp.where(better, barg, idx_ref[...])

    is_last = i == NBLK - 1

    @pl.when(jnp.logical_not(is_last))
    def _acc():
        scores_ref[...] = newv

    @pl.when(is_last)
    def _fin():
        qsq = jnp.sum(q * q, axis=1, keepdims=True)
        ps = jnp.sqrt(jnp.maximum(newv + qsq, 0.0) + 1e-12)
        scores_ref[...] = ps
        img_ref[...] = jnp.max(ps, axis=(0, 1), keepdims=True)


def kernel(queries, keys):
    scores, img, idx = pl.pallas_call(
        _nn_kernel,
        grid=(NBLK,),
        in_specs=[
            pl.BlockSpec((Q_DIM, D_DIM), lambda i: (0, 0)),
            pl.BlockSpec((KB, D_DIM), lambda i: (i, 0)),
        ],
        out_specs=[
            pl.BlockSpec((Q_DIM, 1), lambda i: (0, 0)),
            pl.BlockSpec((1, 1), lambda i: (0, 0)),
            pl.BlockSpec((Q_DIM, 1), lambda i: (0, 0)),
        ],
        out_shape=[
            jax.ShapeDtypeStruct((Q_DIM, 1), jnp.float32),
            jax.ShapeDtypeStruct((1, 1), jnp.float32),
            jax.ShapeDtypeStruct((Q_DIM, 1), jnp.int32),
        ],
        compiler_params=pltpu.CompilerParams(
            dimension_semantics=("arbitrary",),
        ),
    )(queries, keys)
    return scores[:, 0], img[0, 0], idx[:, 0]
